# baseline (speedup 1.0000x reference)
import functools

import jax
import jax.numpy as jnp
from jax import lax
from jax.experimental import pallas as pl
from jax.experimental.pallas import tpu as pltpu
from jax.experimental.pallas import tpu_sc as plsc

N_ATOMS = 50000
N_PAIRS = 1600000
NC, NS = 2, 16
NW = NC * NS
PAIRS_PER_W = N_PAIRS // NW
SB = 2000
N_SB = PAIRS_PER_W // SB
CHUNK = 80
N_CHUNK = SB // CHUNK
CN_ROWS, CN_COLS = 512, 128

S6, S8, A1, A2 = 1.0, 1.9889, 0.3708, 5.0572
K1, K3 = 16.0, 4.0
K1K2 = 64.0 / 3.0

_mesh = plsc.VectorSubcoreMesh(core_axis_name="c", subcore_axis_name="s")
_cparams = pltpu.CompilerParams(needs_layout_passes=False,
                                use_tc_tiling_on_sc=False)


def _wid():
    return lax.axis_index("s") * NC + lax.axis_index("c")


@functools.partial(
    pl.kernel,
    mesh=_mesh,
    out_type=[
        jax.ShapeDtypeStruct((NW * N_SB, N_CHUNK, CHUNK), jnp.int32),
        jax.ShapeDtypeStruct((NC, CN_ROWS, CN_COLS), jnp.float32),
    ],
    scratch_types=[
        pltpu.VMEM((N_ATOMS,), jnp.int32),
        pltpu.VMEM((96,), jnp.float32),
        pltpu.VMEM((CN_ROWS, CN_COLS), jnp.float32),
        pltpu.VMEM((SB,), jnp.int32),
        pltpu.VMEM((SB,), jnp.int32),
        pltpu.VMEM((SB,), jnp.float32),
        pltpu.VMEM((N_CHUNK, CHUNK), jnp.int32),
        pltpu.VMEM((4, 128), jnp.int32),
        pltpu.VMEM_SHARED((CN_ROWS, CN_COLS), jnp.float32),
    ],
    compiler_params=_cparams,
)
def _phase1(i_hbm, j_hbm, d_hbm, sp_hbm, rad_hbm, zer_hbm, idr_hbm,
            keys_hbm, cnp_hbm,
            sp_v, rad_v, cn_v, i_v, j_v, d_v, key_v, idr_v, shared):
    cid = lax.axis_index("c")
    sid = lax.axis_index("s")
    w = _wid()
    base = w * PAIRS_PER_W

    pltpu.sync_copy(sp_hbm, sp_v)
    pltpu.sync_copy(rad_hbm, rad_v)
    pltpu.sync_copy(zer_hbm, cn_v)
    pltpu.sync_copy(idr_hbm, idr_v)

    def super_block(sb, _):
        off = base + sb * SB
        pltpu.sync_copy(i_hbm.at[pl.ds(off, SB)], i_v)
        pltpu.sync_copy(j_hbm.at[pl.ds(off, SB)], j_v)
        pltpu.sync_copy(d_hbm.at[pl.ds(off, SB)], d_v)

        def group(g, _):
            i16 = i_v[pl.ds(g * 16, 16)]
            j16 = j_v[pl.ds(g * 16, 16)]
            d16 = d_v[pl.ds(g * 16, 16)]
            spi = plsc.load_gather(sp_v, [i16])
            spj = plsc.load_gather(sp_v, [j16])
            key = spi * 95 + spj
            key_v[g // 5, pl.ds((g % 5) * 16, 16)] = key
            rc = plsc.load_gather(rad_v, [spi]) + plsc.load_gather(rad_v, [spj])
            cf = 1.0 / (1.0 + jnp.exp(K1 - K1K2 * rc / d16))
            plsc.addupdate_scatter(
                cn_v, [lax.shift_right_logical(i16, 7), lax.bitwise_and(i16, 127)], cf)
            plsc.addupdate_scatter(
                cn_v, [lax.shift_right_logical(j16, 7), lax.bitwise_and(j16, 127)], cf)
            return 0

        lax.fori_loop(0, SB // 16, group, 0)
        pltpu.sync_copy(key_v, keys_hbm.at[w * N_SB + sb])
        return 0

    lax.fori_loop(0, N_SB, super_block, 0)

    @pl.when(sid == 0)
    def _():
        pltpu.sync_copy(cn_v, shared)

    plsc.subcore_barrier()

    @pl.when(sid != 0)
    def _():
        for c in range(4):
            pltpu.sync_copy(cn_v.at[pl.ds(c * 128, 128)],
                            shared.at[idr_v.at[c]], add=True)

    plsc.subcore_barrier()

    @pl.when(sid == 0)
    def _():
        pltpu.sync_copy(shared, cnp_hbm.at[cid])


@functools.partial(
    pl.kernel,
    mesh=_mesh,
    out_type=jax.ShapeDtypeStruct((NW, 16), jnp.float32),
    scratch_types=[
        pltpu.VMEM((CN_ROWS, CN_COLS), jnp.float32),
        pltpu.VMEM((SB,), jnp.int32),
        pltpu.VMEM((SB,), jnp.int32),
        pltpu.VMEM((SB,), jnp.float32),
        pltpu.VMEM((N_CHUNK, CHUNK), jnp.int32),
        pltpu.VMEM((SB,), jnp.float32),
        pltpu.VMEM((SB,), jnp.float32),
        pltpu.VMEM((CHUNK, 48), jnp.int32),
        pltpu.VMEM((CHUNK, 48), jnp.int32),
        pltpu.VMEM((CN_ROWS // 16, CN_COLS), jnp.float32),
        pltpu.VMEM((CN_ROWS // 16, CN_COLS), jnp.float32),
        pltpu.VMEM((16,), jnp.float32),
        pltpu.SemaphoreType.DMA,
        pltpu.SemaphoreType.DMA,
        pltpu.VMEM_SHARED((CN_ROWS, CN_COLS), jnp.float32),
    ],
    compiler_params=_cparams,
)
def _phase2(i_hbm, j_hbm, d_hbm, keys_hbm, cnp_hbm, tab_hbm,
            out_hbm,
            cn_v, i_v, j_v, d_v, key_v, cni_v, cnj_v, rows_a, rows_b, pa_v,
            pb_v, acc_v, sem_a, sem_b, shared):
    sid = lax.axis_index("s")
    w = _wid()
    base = w * PAIRS_PER_W
    rpt = CN_ROWS // 16

    pltpu.sync_copy(cnp_hbm.at[0].at[pl.ds(sid * rpt, rpt)], pa_v)
    pltpu.sync_copy(cnp_hbm.at[1].at[pl.ds(sid * rpt, rpt)], pb_v)
    for r in range(rpt):
        for k in range(CN_COLS // 16):
            sl = pl.ds(k * 16, 16)
            pa_v[r, sl] = pa_v[r, sl] + pb_v[r, sl]
    pltpu.sync_copy(pa_v, shared.at[pl.ds(sid * rpt, rpt)])
    plsc.subcore_barrier()
    pltpu.sync_copy(shared, cn_v)

    zero16 = jnp.zeros((16,), jnp.float32)
    acc_v[...] = zero16
    iota16 = lax.iota(jnp.int32, 16)

    def super_block(sb, _):
        off = base + sb * SB
        pltpu.sync_copy(i_hbm.at[pl.ds(off, SB)], i_v)
        pltpu.sync_copy(j_hbm.at[pl.ds(off, SB)], j_v)
        pltpu.sync_copy(d_hbm.at[pl.ds(off, SB)], d_v)
        pltpu.sync_copy(keys_hbm.at[w * N_SB + sb], key_v)
        pltpu.async_copy(tab_hbm.at[key_v.at[0]], rows_a, sem_a)

        def gather_cn(g, _):
            i16 = i_v[pl.ds(g * 16, 16)]
            j16 = j_v[pl.ds(g * 16, 16)]
            cni_v[pl.ds(g * 16, 16)] = plsc.load_gather(
                cn_v, [lax.shift_right_logical(i16, 7), lax.bitwise_and(i16, 127)])
            cnj_v[pl.ds(g * 16, 16)] = plsc.load_gather(
                cn_v, [lax.shift_right_logical(j16, 7), lax.bitwise_and(j16, 127)])
            return 0

        lax.fori_loop(0, SB // 16, gather_cn, 0)

        def compute(c, rows_v):
            ng = CHUNK // 16
            cni_t, cnj_t, d_t, pidx_t = [], [], [], []
            for t in range(ng):
                sl = pl.ds(c * CHUNK + t * 16, 16)
                cni_t.append(cni_v[sl])
                cnj_t.append(cnj_v[sl])
                d_t.append(d_v[sl])
                pidx_t.append(iota16 + (t * 16))

            himask = jnp.full((16,), -65536, jnp.int32)

            def gbody(gp, carry):
                c0 = jnp.broadcast_to(gp * 2, (16,)).astype(jnp.int32)
                c1 = c0 + 1
                cc = jnp.broadcast_to(gp + 26, (16,)).astype(jnp.int32)
                new = []
                for t in range(ng):
                    wsum, zsum = carry[2 * t], carry[2 * t + 1]
                    wa = plsc.load_gather(rows_v, [pidx_t[t], c0])
                    wb = plsc.load_gather(rows_v, [pidx_t[t], c1])
                    wc = plsc.load_gather(rows_v, [pidx_t[t], cc])
                    ra_a = plsc.bitcast(lax.shift_left(wa, 16), jnp.float32)
                    rb_a = plsc.bitcast(lax.bitwise_and(wa, himask), jnp.float32)
                    ra_b = plsc.bitcast(lax.shift_left(wb, 16), jnp.float32)
                    rb_b = plsc.bitcast(lax.bitwise_and(wb, himask), jnp.float32)
                    c6_a = plsc.bitcast(lax.shift_left(wc, 16), jnp.float32)
                    c6_b = plsc.bitcast(lax.bitwise_and(wc, himask), jnp.float32)
                    dia = cni_t[t] - ra_a
                    dja = cnj_t[t] - rb_a
                    gda = jnp.exp(-K3 * (dia * dia + dja * dja))
                    dib = cni_t[t] - ra_b
                    djb = cnj_t[t] - rb_b
                    gdb = jnp.exp(-K3 * (dib * dib + djb * djb))
                    new.append(wsum + gda + gdb)
                    new.append(zsum + c6_a * gda + c6_b * gdb)
                return tuple(new)

            carry = lax.fori_loop(0, 13, gbody, (zero16,) * (2 * ng))
            total = zero16
            for t in range(ng):
                wsum, zsum = carry[2 * t], carry[2 * t + 1]
                qab = plsc.bitcast(
                    plsc.load_gather(rows_v, [pidx_t[t], jnp.full((16,), 39, jnp.int32)]),
                    jnp.float32)
                r0 = plsc.bitcast(
                    plsc.load_gather(rows_v, [pidx_t[t], jnp.full((16,), 40, jnp.int32)]),
                    jnp.float32)
                c6 = zsum / (wsum + 1e-10)
                c8 = 3.0 * c6 * qab
                b = A1 * r0 + A2
                b2 = b * b
                b6 = b2 * b2 * b2
                b8 = b6 * b2
                d16 = d_t[t]
                d2 = d16 * d16
                d6 = d2 * d2 * d2
                d8 = d6 * d2
                total = total + S6 * c6 / (d6 + b6) + S8 * c8 / (d8 + b8)
            acc_v[...] = acc_v[...] + total

        pltpu.make_async_copy(tab_hbm.at[key_v.at[0]], rows_a, sem_a).wait()

        def chunk(c, _):
            compute(c, rows_a)
            pltpu.async_copy(tab_hbm.at[key_v.at[c + 1]], rows_a, sem_a).wait()
            return 0

        lax.fori_loop(0, N_CHUNK - 1, chunk, 0)
        compute(N_CHUNK - 1, rows_a)
        return 0

    lax.fori_loop(0, N_SB, super_block, 0)
    pltpu.sync_copy(acc_v, out_hbm.at[w])


def kernel(species, energies, atom_index12, distances, covalent_radii,
           cutoff_radii, sqrt_q_ab, c6ref, cn_a_ref, cn_b_ref):
    sp = species.reshape(-1).astype(jnp.int32)
    i_arr = atom_index12[0].astype(jnp.int32)
    j_arr = atom_index12[1].astype(jnp.int32)
    rad96 = jnp.pad(covalent_radii.astype(jnp.float32), (0, 1))
    nk = 95 * 95

    def packw(lo, hi):
        lo16 = lax.bitcast_convert_type(lo.astype(jnp.bfloat16), jnp.uint16)
        hi16 = lax.bitcast_convert_type(hi.astype(jnp.bfloat16), jnp.uint16)
        w32 = (hi16.astype(jnp.uint32) << 16) | lo16.astype(jnp.uint32)
        return lax.bitcast_convert_type(w32, jnp.int32)

    sent = jnp.full((nk, 1), 3e4, jnp.float32)
    ra26 = jnp.concatenate([cn_a_ref.reshape(nk, 25), sent], axis=1)
    rb26 = jnp.concatenate([cn_b_ref.reshape(nk, 25), sent], axis=1)
    c626 = jnp.concatenate([c6ref.reshape(nk, 25),
                            jnp.zeros((nk, 1), jnp.float32)], axis=1)
    tab = jnp.concatenate([
        packw(ra26, rb26),
        packw(c626[:, 0::2], c626[:, 1::2]),
        lax.bitcast_convert_type(sqrt_q_ab.reshape(nk, 1), jnp.int32),
        lax.bitcast_convert_type(cutoff_radii.reshape(nk, 1), jnp.int32),
        jnp.zeros((nk, 7), jnp.int32)], axis=1)
    zer = jnp.zeros((CN_ROWS, CN_COLS), jnp.float32)
    idr = jnp.arange(CN_ROWS, dtype=jnp.int32).reshape(4, 128)
    keys, cnp = _phase1(i_arr, j_arr, distances, sp, rad96, zer, idr)
    psum = _phase2(i_arr, j_arr, distances, keys, cnp, tab)
    return (species, energies - jnp.sum(psum))


# speedup vs baseline: 3.7111x; 3.7111x over previous
import functools

import jax
import jax.numpy as jnp
from jax import lax
from jax.experimental import pallas as pl
from jax.experimental.pallas import tpu as pltpu
from jax.experimental.pallas import tpu_sc as plsc

N_ATOMS = 50000
N_PAIRS = 1600000
NC, NS = 2, 16
NW = NC * NS
PAIRS_PER_W = N_PAIRS // NW
SB = 2000
N_SB = PAIRS_PER_W // SB
CHUNK = 80
N_CHUNK = SB // CHUNK
CN_ROWS, CN_COLS = 512, 128

S6, S8, A1, A2 = 1.0, 1.9889, 0.3708, 5.0572
K1, K3 = 16.0, 4.0
K1K2 = 64.0 / 3.0

_mesh = plsc.VectorSubcoreMesh(core_axis_name="c", subcore_axis_name="s")
_cparams = pltpu.CompilerParams(needs_layout_passes=False,
                                use_tc_tiling_on_sc=False)


def _wid():
    return lax.axis_index("s") * NC + lax.axis_index("c")


@functools.partial(
    pl.kernel,
    mesh=_mesh,
    out_type=[
        jax.ShapeDtypeStruct((NW * N_SB, N_CHUNK, CHUNK), jnp.int32),
        jax.ShapeDtypeStruct((NC, CN_ROWS, CN_COLS), jnp.float32),
    ],
    scratch_types=[
        pltpu.VMEM((N_ATOMS,), jnp.int32),
        pltpu.VMEM((96,), jnp.float32),
        pltpu.VMEM((CN_ROWS, CN_COLS), jnp.float32),
        pltpu.VMEM((SB,), jnp.int32),
        pltpu.VMEM((SB,), jnp.int32),
        pltpu.VMEM((SB,), jnp.float32),
        pltpu.VMEM((N_CHUNK, CHUNK), jnp.int32),
        pltpu.VMEM((4, 128), jnp.int32),
        pltpu.VMEM_SHARED((CN_ROWS, CN_COLS), jnp.float32),
    ],
    compiler_params=_cparams,
)
def _phase1(i_hbm, j_hbm, d_hbm, sp_hbm, rad_hbm, zer_hbm, idr_hbm,
            keys_hbm, cnp_hbm,
            sp_v, rad_v, cn_v, i_v, j_v, d_v, key_v, idr_v, shared):
    cid = lax.axis_index("c")
    sid = lax.axis_index("s")
    w = _wid()
    base = w * PAIRS_PER_W

    pltpu.sync_copy(sp_hbm, sp_v)
    pltpu.sync_copy(rad_hbm, rad_v)
    pltpu.sync_copy(zer_hbm, cn_v)
    pltpu.sync_copy(idr_hbm, idr_v)

    def super_block(sb, _):
        off = base + sb * SB
        pltpu.sync_copy(i_hbm.at[pl.ds(off, SB)], i_v)
        pltpu.sync_copy(j_hbm.at[pl.ds(off, SB)], j_v)
        pltpu.sync_copy(d_hbm.at[pl.ds(off, SB)], d_v)

        def group(g, _):
            i16 = i_v[pl.ds(g * 16, 16)]
            j16 = j_v[pl.ds(g * 16, 16)]
            d16 = d_v[pl.ds(g * 16, 16)]
            spi = plsc.load_gather(sp_v, [i16])
            spj = plsc.load_gather(sp_v, [j16])
            key = spi * 95 + spj
            key_v[g // 5, pl.ds((g % 5) * 16, 16)] = key
            rc = plsc.load_gather(rad_v, [spi]) + plsc.load_gather(rad_v, [spj])
            cf = 1.0 / (1.0 + jnp.exp(K1 - K1K2 * rc / d16))
            plsc.addupdate_scatter(
                cn_v, [lax.shift_right_logical(i16, 7), lax.bitwise_and(i16, 127)], cf)
            plsc.addupdate_scatter(
                cn_v, [lax.shift_right_logical(j16, 7), lax.bitwise_and(j16, 127)], cf)
            return 0

        lax.fori_loop(0, SB // 16, group, 0)
        pltpu.sync_copy(key_v, keys_hbm.at[w * N_SB + sb])
        return 0

    lax.fori_loop(0, N_SB, super_block, 0)

    @pl.when(sid == 0)
    def _():
        pltpu.sync_copy(cn_v, shared)

    plsc.subcore_barrier()

    @pl.when(sid != 0)
    def _():
        for c in range(4):
            pltpu.sync_copy(cn_v.at[pl.ds(c * 128, 128)],
                            shared.at[idr_v.at[c]], add=True)

    plsc.subcore_barrier()

    @pl.when(sid == 0)
    def _():
        pltpu.sync_copy(shared, cnp_hbm.at[cid])


@functools.partial(
    pl.kernel,
    mesh=_mesh,
    out_type=jax.ShapeDtypeStruct((NW, 16), jnp.float32),
    scratch_types=[
        pltpu.VMEM((CN_ROWS, CN_COLS), jnp.float32),
        pltpu.VMEM((SB,), jnp.int32),
        pltpu.VMEM((SB,), jnp.int32),
        pltpu.VMEM((SB,), jnp.float32),
        pltpu.VMEM((N_CHUNK, CHUNK), jnp.int32),
        pltpu.VMEM((SB + CHUNK,), jnp.int32),
        pltpu.VMEM((SB + CHUNK,), jnp.float32),
        pltpu.VMEM((SB + CHUNK,), jnp.float32),
        pltpu.VMEM((SB + CHUNK,), jnp.float32),
        pltpu.VMEM((CHUNK, 48), jnp.int32),
        pltpu.VMEM((CN_ROWS // 16, CN_COLS), jnp.float32),
        pltpu.VMEM((CN_ROWS // 16, CN_COLS), jnp.float32),
        pltpu.VMEM((16,), jnp.float32),
        pltpu.SemaphoreType.DMA,
        pltpu.VMEM_SHARED((CN_ROWS, CN_COLS), jnp.float32),
    ],
    compiler_params=_cparams,
)
def _phase2(i_hbm, j_hbm, d_hbm, keys_hbm, cnp_hbm, tab_hbm,
            out_hbm,
            cn_v, i_v, j_v, d_v, key_v, key_c, cni_c, cnj_c, d_c, rows_a,
            pa_v, pb_v, acc_v, sem_a, shared):
    sid = lax.axis_index("s")
    w = _wid()
    base = w * PAIRS_PER_W
    rpt = CN_ROWS // 16

    pltpu.sync_copy(cnp_hbm.at[0].at[pl.ds(sid * rpt, rpt)], pa_v)
    pltpu.sync_copy(cnp_hbm.at[1].at[pl.ds(sid * rpt, rpt)], pb_v)
    for r in range(rpt):
        for k in range(CN_COLS // 16):
            sl = pl.ds(k * 16, 16)
            pa_v[r, sl] = pa_v[r, sl] + pb_v[r, sl]
    pltpu.sync_copy(pa_v, shared.at[pl.ds(sid * rpt, rpt)])
    plsc.subcore_barrier()
    pltpu.sync_copy(shared, cn_v)

    zero16 = jnp.zeros((16,), jnp.float32)
    acc_v[...] = zero16
    iota16 = lax.iota(jnp.int32, 16)

    alive_t = jnp.full((16,), 13.5, jnp.float32)

    def super_block(sb, _):
        off = base + sb * SB
        pltpu.sync_copy(i_hbm.at[pl.ds(off, SB)], i_v)
        pltpu.sync_copy(j_hbm.at[pl.ds(off, SB)], j_v)
        pltpu.sync_copy(d_hbm.at[pl.ds(off, SB)], d_v)
        pltpu.sync_copy(keys_hbm.at[w * N_SB + sb], key_v)

        def compact(g, na):
            i16 = i_v[pl.ds(g * 16, 16)]
            j16 = j_v[pl.ds(g * 16, 16)]
            cni = plsc.load_gather(
                cn_v, [lax.shift_right_logical(i16, 7), lax.bitwise_and(i16, 127)])
            cnj = plsc.load_gather(
                cn_v, [lax.shift_right_logical(j16, 7), lax.bitwise_and(j16, 127)])
            alive = jnp.logical_and(cni <= alive_t, cnj <= alive_t)
            key16 = key_v[g // 5, pl.ds((g % 5) * 16, 16)]
            d16 = d_v[pl.ds(g * 16, 16)]
            plsc.store_compressed(key_c.at[pl.ds(na, 16)], key16, mask=alive)
            plsc.store_compressed(cni_c.at[pl.ds(na, 16)], cni, mask=alive)
            plsc.store_compressed(cnj_c.at[pl.ds(na, 16)], cnj, mask=alive)
            plsc.store_compressed(d_c.at[pl.ds(na, 16)], d16, mask=alive)
            cnt = plsc.all_reduce_population_count(alive)
            return na + jnp.max(cnt)

        na = lax.fori_loop(0, SB // 16, compact, jnp.int32(0))
        zkey = jnp.zeros((16,), jnp.int32)
        for k in range(CHUNK // 16):
            key_c[pl.ds(na + k * 16, 16)] = zkey

        def compute(c, rows_v):
            ng = CHUNK // 16
            cni_t, cnj_t, d_t, pidx_t, valid_t = [], [], [], [], []
            for t in range(ng):
                sl = pl.ds(c * CHUNK + t * 16, 16)
                cni_t.append(cni_c[sl])
                cnj_t.append(cnj_c[sl])
                d_t.append(d_c[sl])
                pidx_t.append(iota16 + (t * 16))
                valid_t.append((c * CHUNK + t * 16) + iota16 < na)

            himask = jnp.full((16,), -65536, jnp.int32)

            def gbody(gp, carry):
                c0 = jnp.broadcast_to(gp * 2, (16,)).astype(jnp.int32)
                c1 = c0 + 1
                cc = jnp.broadcast_to(gp + 26, (16,)).astype(jnp.int32)
                new = []
                for t in range(ng):
                    wsum, zsum = carry[2 * t], carry[2 * t + 1]
                    wa = plsc.load_gather(rows_v, [pidx_t[t], c0])
                    wb = plsc.load_gather(rows_v, [pidx_t[t], c1])
                    wc = plsc.load_gather(rows_v, [pidx_t[t], cc])
                    ra_a = plsc.bitcast(lax.shift_left(wa, 16), jnp.float32)
                    rb_a = plsc.bitcast(lax.bitwise_and(wa, himask), jnp.float32)
                    ra_b = plsc.bitcast(lax.shift_left(wb, 16), jnp.float32)
                    rb_b = plsc.bitcast(lax.bitwise_and(wb, himask), jnp.float32)
                    c6_a = plsc.bitcast(lax.shift_left(wc, 16), jnp.float32)
                    c6_b = plsc.bitcast(lax.bitwise_and(wc, himask), jnp.float32)
                    dia = cni_t[t] - ra_a
                    dja = cnj_t[t] - rb_a
                    gda = jnp.exp(-K3 * (dia * dia + dja * dja))
                    dib = cni_t[t] - ra_b
                    djb = cnj_t[t] - rb_b
                    gdb = jnp.exp(-K3 * (dib * dib + djb * djb))
                    new.append(wsum + gda + gdb)
                    new.append(zsum + c6_a * gda + c6_b * gdb)
                return tuple(new)

            carry = lax.fori_loop(0, 13, gbody, (zero16,) * (2 * ng))
            total = zero16
            for t in range(ng):
                wsum, zsum = carry[2 * t], carry[2 * t + 1]
                qab = plsc.bitcast(
                    plsc.load_gather(rows_v, [pidx_t[t], jnp.full((16,), 39, jnp.int32)]),
                    jnp.float32)
                r0 = plsc.bitcast(
                    plsc.load_gather(rows_v, [pidx_t[t], jnp.full((16,), 40, jnp.int32)]),
                    jnp.float32)
                c6 = zsum / (wsum + 1e-10)
                c8 = 3.0 * c6 * qab
                b = A1 * r0 + A2
                b2 = b * b
                b6 = b2 * b2 * b2
                b8 = b6 * b2
                d16 = d_t[t]
                d2 = d16 * d16
                d6 = d2 * d2 * d2
                d8 = d6 * d2
                term = S6 * c6 / (d6 + b6) + S8 * c8 / (d8 + b8)
                total = total + jnp.where(valid_t[t], term, 0.0)
            acc_v[...] = acc_v[...] + total

        nch = (na + (CHUNK - 1)) // CHUNK

        def chunk(c, _):
            pltpu.async_copy(
                tab_hbm.at[key_c.at[pl.ds(c * CHUNK, CHUNK)]], rows_a,
                sem_a).wait()
            compute(c, rows_a)
            return 0

        lax.fori_loop(0, nch, chunk, 0)
        return 0

    lax.fori_loop(0, N_SB, super_block, 0)
    pltpu.sync_copy(acc_v, out_hbm.at[w])


def kernel(species, energies, atom_index12, distances, covalent_radii,
           cutoff_radii, sqrt_q_ab, c6ref, cn_a_ref, cn_b_ref):
    sp = species.reshape(-1).astype(jnp.int32)
    i_arr = atom_index12[0].astype(jnp.int32)
    j_arr = atom_index12[1].astype(jnp.int32)
    rad96 = jnp.pad(covalent_radii.astype(jnp.float32), (0, 1))
    nk = 95 * 95

    def packw(lo, hi):
        lo16 = lax.bitcast_convert_type(lo.astype(jnp.bfloat16), jnp.uint16)
        hi16 = lax.bitcast_convert_type(hi.astype(jnp.bfloat16), jnp.uint16)
        w32 = (hi16.astype(jnp.uint32) << 16) | lo16.astype(jnp.uint32)
        return lax.bitcast_convert_type(w32, jnp.int32)

    sent = jnp.full((nk, 1), 3e4, jnp.float32)
    ra26 = jnp.concatenate([cn_a_ref.reshape(nk, 25), sent], axis=1)
    rb26 = jnp.concatenate([cn_b_ref.reshape(nk, 25), sent], axis=1)
    c626 = jnp.concatenate([c6ref.reshape(nk, 25),
                            jnp.zeros((nk, 1), jnp.float32)], axis=1)
    tab = jnp.concatenate([
        packw(ra26, rb26),
        packw(c626[:, 0::2], c626[:, 1::2]),
        lax.bitcast_convert_type(sqrt_q_ab.reshape(nk, 1), jnp.int32),
        lax.bitcast_convert_type(cutoff_radii.reshape(nk, 1), jnp.int32),
        jnp.zeros((nk, 7), jnp.int32)], axis=1)
    zer = jnp.zeros((CN_ROWS, CN_COLS), jnp.float32)
    idr = jnp.arange(CN_ROWS, dtype=jnp.int32).reshape(4, 128)
    keys, cnp = _phase1(i_arr, j_arr, distances, sp, rad96, zer, idr)
    psum = _phase2(i_arr, j_arr, distances, keys, cnp, tab)
    return (species, energies - jnp.sum(psum))
